# baseline (speedup 1.0000x reference)
import functools
from math import sqrt

import jax
import jax.numpy as jnp
from jax import lax
from jax.experimental import pallas as pl
from jax.experimental.pallas import tpu as pltpu
from jax.experimental.pallas import tpu_sc as plsc

D_MODEL = 1024
SCALE = sqrt(D_MODEL)
NW = 32
LANES = 16
CHUNK = 32


def _make_sc_kernel(n_rows, seq):
    b_per_w = n_rows // NW
    n_chunks = b_per_w // CHUNK
    mesh = plsc.VectorSubcoreMesh(core_axis_name="c", subcore_axis_name="s")

    @functools.partial(
        pl.kernel,
        out_type=jax.ShapeDtypeStruct((n_rows, D_MODEL), jnp.float32),
        mesh=mesh,
        scratch_types=[
            pltpu.VMEM((b_per_w,), jnp.int32),
            pltpu.VMEM((CHUNK, D_MODEL), jnp.float32),
            pltpu.VMEM((CHUNK, D_MODEL), jnp.float32),
            pltpu.SemaphoreType.DMA,
        ],
    )
    def emb_kernel(x_hbm, table_hbm, pe_hbm, out_hbm, idx_v, rows_v, pe_v, sem):
        wid = lax.axis_index("s") * 2 + lax.axis_index("c")
        base = wid * b_per_w
        s_base = base % seq
        pltpu.sync_copy(x_hbm.at[pl.ds(base, b_per_w)], idx_v)

        def chunk_body(c, carry):
            off = c * CHUNK
            pltpu.sync_copy(pe_hbm.at[pl.ds(s_base + off, CHUNK)], pe_v)
            pltpu.async_copy(
                table_hbm.at[idx_v.at[pl.ds(off, CHUNK)]], rows_v, sem
            ).wait()

            def row_body(r, carry):
                def col_body(j, carry):
                    sl = pl.ds(j * LANES, LANES)
                    rows_v[r, sl] = rows_v[r, sl] * SCALE + pe_v[r, sl]
                    return carry

                return lax.fori_loop(0, D_MODEL // LANES, col_body, carry)

            lax.fori_loop(0, CHUNK, row_body, 0)
            pltpu.sync_copy(rows_v, out_hbm.at[pl.ds(base + off, CHUNK)])
            return carry

        lax.fori_loop(0, n_chunks, chunk_body, 0)

    return emb_kernel


@jax.jit
def kernel(x, embed_table, pe):
    batch, seq = x.shape
    n_rows = batch * seq
    x_flat = x.reshape(n_rows).astype(jnp.int32)
    pe2d = pe.reshape(pe.shape[1], pe.shape[2])[:seq]
    out = _make_sc_kernel(n_rows, seq)(x_flat, embed_table, pe2d)
    return out.reshape(batch, seq, D_MODEL)


# speedup vs baseline: 1.1252x; 1.1252x over previous
import functools
from math import sqrt

import jax
import jax.numpy as jnp
from jax import lax
from jax.experimental import pallas as pl
from jax.experimental.pallas import tpu as pltpu
from jax.experimental.pallas import tpu_sc as plsc

D_MODEL = 1024
SCALE = sqrt(D_MODEL)
NW = 32
LANES = 16
PCHUNK = 16


def _make_sc_kernel(batch, seq):
    pos_per_w = seq // NW
    n_chunks = pos_per_w // PCHUNK
    mesh = plsc.VectorSubcoreMesh(core_axis_name="c", subcore_axis_name="s")

    @functools.partial(
        pl.kernel,
        out_type=jax.ShapeDtypeStruct((batch * seq, D_MODEL), jnp.float32),
        mesh=mesh,
        scratch_types=[
            pltpu.VMEM((batch * pos_per_w,), jnp.int32),
            pltpu.VMEM((batch, PCHUNK, D_MODEL), jnp.float32),
            pltpu.VMEM((PCHUNK, D_MODEL), jnp.float32),
            pltpu.SemaphoreType.DMA,
        ],
    )
    def emb_kernel(x_hbm, table_hbm, pe_hbm, out_hbm, idx_v, rows_v, pe_v, sem):
        wid = lax.axis_index("s") * 2 + lax.axis_index("c")
        p0 = wid * pos_per_w

        for b in range(batch):
            pltpu.sync_copy(
                x_hbm.at[pl.ds(b * seq + p0, pos_per_w)],
                idx_v.at[pl.ds(b * pos_per_w, pos_per_w)],
            )

        def chunk_body(c, carry):
            off = c * PCHUNK
            copies = [
                pltpu.async_copy(
                    table_hbm.at[idx_v.at[pl.ds(b * pos_per_w + off, PCHUNK)]],
                    rows_v.at[b],
                    sem,
                )
                for b in range(batch)
            ]
            copies.append(
                pltpu.async_copy(pe_hbm.at[pl.ds(p0 + off, PCHUNK)], pe_v, sem)
            )
            for cp in copies:
                cp.wait()

            def row_body(r, carry):
                def col_body(j, carry2):
                    sl = pl.ds(j * LANES, LANES)
                    pe_reg = pe_v[r, sl]
                    for b in range(batch):
                        rows_v[b, r, sl] = rows_v[b, r, sl] * SCALE + pe_reg
                    return carry2

                return lax.fori_loop(0, D_MODEL // LANES, col_body, carry)

            lax.fori_loop(0, PCHUNK, row_body, 0)

            for b in range(batch):
                pltpu.sync_copy(
                    rows_v.at[b],
                    out_hbm.at[pl.ds(b * seq + p0 + off, PCHUNK)],
                )
            return carry

        lax.fori_loop(0, n_chunks, chunk_body, 0)

    return emb_kernel


@jax.jit
def kernel(x, embed_table, pe):
    batch, seq = x.shape
    x_flat = x.reshape(batch * seq).astype(jnp.int32)
    pe2d = pe.reshape(pe.shape[1], pe.shape[2])[:seq]
    out = _make_sc_kernel(batch, seq)(x_flat, embed_table, pe2d)
    return out.reshape(batch, seq, D_MODEL)


# speedup vs baseline: 2.3937x; 2.1274x over previous
import functools
from math import sqrt

import jax
import jax.numpy as jnp
from jax import lax
from jax.experimental import pallas as pl
from jax.experimental.pallas import tpu as pltpu
from jax.experimental.pallas import tpu_sc as plsc

D_MODEL = 1024
SCALE = sqrt(D_MODEL)
NW = 32
LANES = 16
PCHUNK = 8


def _make_sc_kernel(batch, seq):
    pos_per_w = seq // NW
    n_chunks = pos_per_w // PCHUNK
    mesh = plsc.VectorSubcoreMesh(core_axis_name="c", subcore_axis_name="s")

    @functools.partial(
        pl.kernel,
        out_type=jax.ShapeDtypeStruct((batch * seq, D_MODEL), jnp.float32),
        mesh=mesh,
        scratch_types=[
            pltpu.VMEM((batch * pos_per_w,), jnp.int32),
            pltpu.VMEM((batch, PCHUNK, D_MODEL), jnp.float32),
            pltpu.VMEM((batch, PCHUNK, D_MODEL), jnp.float32),
            pltpu.VMEM((PCHUNK, D_MODEL), jnp.float32),
            pltpu.SemaphoreType.DMA,
        ],
    )
    def emb_kernel(
        x_hbm, table_hbm, pe_hbm, out_hbm, idx_v, rows_v, out_v, pe_v, sem
    ):
        wid = lax.axis_index("s") * 2 + lax.axis_index("c")
        p0 = wid * pos_per_w

        for b in range(batch):
            pltpu.sync_copy(
                x_hbm.at[pl.ds(b * seq + p0, pos_per_w)],
                idx_v.at[pl.ds(b * pos_per_w, pos_per_w)],
            )

        def chunk_body(c, carry):
            off = c * PCHUNK
            copies = [
                pltpu.async_copy(
                    table_hbm.at[idx_v.at[pl.ds(b * pos_per_w + off, PCHUNK)]],
                    rows_v.at[b],
                    sem,
                )
                for b in range(batch)
            ]
            copies.append(
                pltpu.async_copy(pe_hbm.at[pl.ds(p0 + off, PCHUNK)], pe_v, sem)
            )
            for cp in copies:
                cp.wait()

            def row_body(r, carry):
                def col_body(j, carry2):
                    sl = pl.ds(j * LANES, LANES)
                    pe_reg = pe_v[r, sl]
                    for b in range(batch):
                        out_v[b, r, sl] = rows_v[b, r, sl] * SCALE + pe_reg
                    return carry2

                return lax.fori_loop(
                    0, D_MODEL // LANES, col_body, carry, unroll=8
                )

            lax.fori_loop(0, PCHUNK, row_body, 0)

            for b in range(batch):
                pltpu.sync_copy(
                    out_v.at[b],
                    out_hbm.at[pl.ds(b * seq + p0 + off, PCHUNK)],
                )
            return carry

        lax.fori_loop(0, n_chunks, chunk_body, 0)

    return emb_kernel


@jax.jit
def kernel(x, embed_table, pe):
    batch, seq = x.shape
    x_flat = x.reshape(batch * seq).astype(jnp.int32)
    pe2d = pe.reshape(pe.shape[1], pe.shape[2])[:seq]
    out = _make_sc_kernel(batch, seq)(x_flat, embed_table, pe2d)
    return out.reshape(batch, seq, D_MODEL)


# speedup vs baseline: 3.0128x; 1.2586x over previous
from math import sqrt

import jax
import jax.numpy as jnp
from jax import lax
from jax.experimental import pallas as pl
from jax.experimental.pallas import tpu as pltpu
from jax.experimental.pallas import tpu_sc as plsc

D_MODEL = 1024
SCALE = sqrt(D_MODEL)
NW = 32
LANES = 16
PCHUNK = 8


def _make_sc_kernel(batch, seq):
    pos_per_w = seq // NW
    n_chunks = pos_per_w // PCHUNK
    mesh = plsc.VectorSubcoreMesh(core_axis_name="c", subcore_axis_name="s")

    @pl.kernel(
        out_type=jax.ShapeDtypeStruct((batch * seq, D_MODEL), jnp.float32),
        mesh=mesh,
        scratch_types=[
            pltpu.VMEM((batch * pos_per_w,), jnp.int32),
            pltpu.VMEM((batch, PCHUNK, D_MODEL), jnp.float32),
            pltpu.VMEM((batch, PCHUNK, D_MODEL), jnp.float32),
            pltpu.VMEM((PCHUNK, D_MODEL), jnp.float32),
            pltpu.VMEM((PCHUNK, D_MODEL), jnp.float32),
            pltpu.SemaphoreType.DMA,
            pltpu.SemaphoreType.DMA,
            pltpu.SemaphoreType.DMA,
        ],
    )
    def emb_kernel(
        x_hbm, table_hbm, pe_hbm, out_hbm,
        idx_v, rows0, rows1, pe0, pe1, sem0, sem1, sem_out,
    ):
        rows = (rows0, rows1)
        pes = (pe0, pe1)
        sems = (sem0, sem1)

        wid = lax.axis_index("s") * 2 + lax.axis_index("c")
        p0 = wid * pos_per_w

        for b in range(batch):
            pltpu.sync_copy(
                x_hbm.at[pl.ds(b * seq + p0, pos_per_w)],
                idx_v.at[pl.ds(b * pos_per_w, pos_per_w)],
            )

        def fire_in(c, k):
            off = c * PCHUNK
            cps = [
                pltpu.async_copy(
                    table_hbm.at[idx_v.at[pl.ds(b * pos_per_w + off, PCHUNK)]],
                    rows[k].at[b],
                    sems[k],
                )
                for b in range(batch)
            ]
            cps.append(
                pltpu.async_copy(pe_hbm.at[pl.ds(p0 + off, PCHUNK)], pes[k], sems[k])
            )
            return cps

        def fire_stores(c, k):
            off = c * PCHUNK
            return [
                pltpu.async_copy(
                    rows[k].at[b],
                    out_hbm.at[pl.ds(b * seq + p0 + off, PCHUNK)],
                    sem_out,
                )
                for b in range(batch)
            ]

        def compute(k):
            def row_body(r, carry):
                def col_body(j, carry2):
                    sl = pl.ds(j * LANES, LANES)
                    pe_reg = pes[k][r, sl]
                    for b in range(batch):
                        rows[k][b, r, sl] = rows[k][b, r, sl] * SCALE + pe_reg
                    return carry2

                return lax.fori_loop(
                    0, D_MODEL // LANES, col_body, carry, unroll=8
                )

            lax.fori_loop(0, PCHUNK, row_body, 0)

        in_flight = fire_in(0, 0)
        store_flight = []
        for c in range(n_chunks):
            k = c & 1
            for cp in in_flight:
                cp.wait()
            for cp in store_flight:
                cp.wait()
            store_flight = []
            if c + 1 < n_chunks:
                in_flight = fire_in(c + 1, 1 - k)
            compute(k)
            store_flight = fire_stores(c, k)
        for cp in store_flight:
            cp.wait()

    return emb_kernel


@jax.jit
def kernel(x, embed_table, pe):
    batch, seq = x.shape
    x_flat = x.reshape(batch * seq).astype(jnp.int32)
    pe2d = pe.reshape(pe.shape[1], pe.shape[2])[:seq]
    out = _make_sc_kernel(batch, seq)(x_flat, embed_table, pe2d)
    return out.reshape(batch, seq, D_MODEL)


# speedup vs baseline: 3.0995x; 1.0288x over previous
from math import sqrt

import jax
import jax.numpy as jnp
from jax import lax
from jax.experimental import pallas as pl
from jax.experimental.pallas import tpu as pltpu
from jax.experimental.pallas import tpu_sc as plsc

D_MODEL = 1024
SCALE = sqrt(D_MODEL)
NW = 32
LANES = 16
PCHUNK = 8


def _make_sc_kernel(batch, seq):
    pos_per_w = seq // NW
    n_chunks = pos_per_w // PCHUNK
    mesh = plsc.VectorSubcoreMesh(core_axis_name="c", subcore_axis_name="s")

    @pl.kernel(
        out_type=jax.ShapeDtypeStruct((batch * seq, D_MODEL), jnp.float32),
        mesh=mesh,
        scratch_types=[
            pltpu.VMEM((batch * pos_per_w,), jnp.int32),
            pltpu.VMEM((batch, PCHUNK, D_MODEL), jnp.float32),
            pltpu.VMEM((batch, PCHUNK, D_MODEL), jnp.float32),
            pltpu.VMEM((PCHUNK, D_MODEL), jnp.float32),
            pltpu.VMEM((PCHUNK, D_MODEL), jnp.float32),
            pltpu.SemaphoreType.DMA,
            pltpu.SemaphoreType.DMA,
            pltpu.SemaphoreType.DMA,
        ],
    )
    def emb_kernel(
        x_hbm, table_hbm, pe_hbm, out_hbm,
        idx_v, rows0, rows1, pe0, pe1, sem0, sem1, sem_out,
    ):
        rows = (rows0, rows1)
        pes = (pe0, pe1)
        sems = (sem0, sem1)

        wid = lax.axis_index("s") * 2 + lax.axis_index("c")
        p0 = wid * pos_per_w

        idx_copies = [
            pltpu.async_copy(
                x_hbm.at[pl.ds(b * seq + p0, pos_per_w)],
                idx_v.at[pl.ds(b * pos_per_w, pos_per_w)],
                sem0,
            )
            for b in range(batch)
        ]
        for cp in idx_copies:
            cp.wait()

        def fire_in(c, k):
            off = c * PCHUNK
            cps = [
                pltpu.async_copy(
                    table_hbm.at[idx_v.at[pl.ds(b * pos_per_w + off, PCHUNK)]],
                    rows[k].at[b],
                    sems[k],
                )
                for b in range(batch)
            ]
            cps.append(
                pltpu.async_copy(pe_hbm.at[pl.ds(p0 + off, PCHUNK)], pes[k], sems[k])
            )
            return cps

        def fire_stores(c, k):
            off = c * PCHUNK
            return [
                pltpu.async_copy(
                    rows[k].at[b],
                    out_hbm.at[pl.ds(b * seq + p0 + off, PCHUNK)],
                    sem_out,
                )
                for b in range(batch)
            ]

        def compute(k):
            def row_body(r, carry):
                def col_body(j, carry2):
                    sl = pl.ds(j * LANES, LANES)
                    pe_reg = pes[k][r, sl]
                    for b in range(batch):
                        rows[k][b, r, sl] = rows[k][b, r, sl] * SCALE + pe_reg
                    return carry2

                return lax.fori_loop(
                    0, D_MODEL // LANES, col_body, carry, unroll=8
                )

            lax.fori_loop(0, PCHUNK, row_body, 0)

        in_flight = fire_in(0, 0)
        store_flight = []
        for c in range(n_chunks):
            k = c & 1
            for cp in in_flight:
                cp.wait()
            for cp in store_flight:
                cp.wait()
            store_flight = []
            if c + 1 < n_chunks:
                in_flight = fire_in(c + 1, 1 - k)
            compute(k)
            store_flight = fire_stores(c, k)
        for cp in store_flight:
            cp.wait()

    return emb_kernel


@jax.jit
def kernel(x, embed_table, pe):
    batch, seq = x.shape
    x_flat = x.reshape(batch * seq).astype(jnp.int32)
    pe2d = pe.reshape(pe.shape[1], pe.shape[2])[:seq]
    out = _make_sc_kernel(batch, seq)(x_flat, embed_table, pe2d)
    return out.reshape(batch, seq, D_MODEL)
